# baseline (speedup 1.0000x reference)
import jax
import jax.numpy as jnp
from jax import lax
from jax.experimental import pallas as pl
from jax.experimental.pallas import tpu as pltpu
from jax.experimental.pallas import tpu_sc as plsc

VOCAB = 100000
DIM = 768
SEQ = 2048
BATCH = 4

NC = 2
NS = 16
LANES = 16
NW = NC * NS
POSB = SEQ // NW
CHUNK = 16
HCHUNK = POSB // CHUNK
NCHUNK = BATCH * HCHUNK
DSLICES = DIM // LANES
NRB = 4


def _body(ids_hbm, vocab_hbm, pos_hbm, out_hbm, idx_v, pos_v,
          r0, r1, r2, r3, psem, gs0, gs1, gs2, gs3, os0, os1, os2, os3):
    rows = [r0, r1, r2, r3]
    gsem = [gs0, gs1, gs2, gs3]
    osem = [os0, os1, os2, os3]

    c_i = lax.axis_index("c")
    s_i = lax.axis_index("s")
    wid = s_i * NC + c_i
    pos_base = wid * POSB

    pload = pltpu.async_copy(pos_hbm.at[pl.ds(pos_base, POSB)], pos_v, psem)
    for b in range(BATCH):
        pltpu.sync_copy(ids_hbm.at[b, pl.ds(pos_base, POSB)], idx_v.at[b])

    g, o = {}, {}

    def start(c):
        b, h = c // HCHUNK, c % HCHUNK
        rb = c % NRB
        g[c] = pltpu.async_copy(
            vocab_hbm.at[idx_v.at[b, pl.ds(h * CHUNK, CHUNK)]],
            rows[rb], gsem[rb])

    start(0)
    start(1)
    pload.wait()
    for c in range(NCHUNK):
        if c + 2 < NCHUNK:
            if c >= 2:
                o[c - 2].wait()
            start(c + 2)
        g[c].wait()
        b, h = c // HCHUNK, c % HCHUNK
        rb = c % NRB

        def row_body(r, _, rb=rb, h=h):
            for d in range(DSLICES):
                sl = pl.ds(d * LANES, LANES)
                plsc.addupdate(rows[rb].at[r, sl], pos_v[h * CHUNK + r, sl])
            return 0

        lax.fori_loop(0, CHUNK, row_body, 0)
        o[c] = pltpu.async_copy(
            rows[rb],
            out_hbm.at[pl.ds(b * SEQ + pos_base + h * CHUNK, CHUNK)],
            osem[rb])
    for c in range(NCHUNK - 4, NCHUNK):
        o[c].wait()


@jax.jit
def kernel(input_ids, vocab_W, pos_W):
    ids = input_ids.astype(jnp.int32)
    mesh = plsc.VectorSubcoreMesh(core_axis_name="c", subcore_axis_name="s")
    run = pl.kernel(
        _body,
        out_type=jax.ShapeDtypeStruct((BATCH * SEQ, DIM), jnp.float32),
        mesh=mesh,
        scratch_types=(
            [pltpu.VMEM((BATCH, POSB), jnp.int32),
             pltpu.VMEM((POSB, DIM), jnp.float32)]
            + [pltpu.VMEM((CHUNK, DIM), jnp.float32) for _ in range(NRB)]
            + [pltpu.SemaphoreType.DMA for _ in range(1 + 2 * NRB)]
        ),
    )
    out = run(ids, vocab_W, pos_W)
    return out.reshape(BATCH, SEQ, DIM)


# speedup vs baseline: 1.3844x; 1.3844x over previous
import jax
import jax.numpy as jnp
from jax import lax
from jax.experimental import pallas as pl
from jax.experimental.pallas import tpu as pltpu
from jax.experimental.pallas import tpu_sc as plsc

VOCAB = 100000
DIM = 768
SEQ = 2048
BATCH = 4

NC = 2
NS = 16
LANES = 16
NW = NC * NS
POSB = SEQ // NW
CHUNK = 16
HCHUNK = POSB // CHUNK
NCHUNK = BATCH * HCHUNK
DSLICES = DIM // LANES
NRB = 4


def _body(ids_hbm, vocab_hbm, pos_hbm, out_hbm, idx_v, pos_v,
          r0, r1, r2, r3, psem, gs0, gs1, gs2, gs3, os0, os1, os2, os3):
    rows = [r0, r1, r2, r3]
    gsem = [gs0, gs1, gs2, gs3]
    osem = [os0, os1, os2, os3]

    c_i = lax.axis_index("c")
    s_i = lax.axis_index("s")
    wid = s_i * NC + c_i
    pos_base = wid * POSB

    pload = pltpu.async_copy(pos_hbm.at[pl.ds(pos_base, POSB)], pos_v, psem)
    for b in range(BATCH):
        pltpu.sync_copy(ids_hbm.at[b, pl.ds(pos_base, POSB)], idx_v.at[b])

    g, o = {}, {}

    def start(c):
        b, h = c // HCHUNK, c % HCHUNK
        rb = c % NRB
        g[c] = pltpu.async_copy(
            vocab_hbm.at[idx_v.at[b, pl.ds(h * CHUNK, CHUNK)]],
            rows[rb], gsem[rb])

    start(0)
    start(1)
    pload.wait()
    for c in range(NCHUNK):
        if c + 2 < NCHUNK:
            if c >= 2:
                o[c - 2].wait()
            start(c + 2)
        g[c].wait()
        b, h = c // HCHUNK, c % HCHUNK
        rb = c % NRB

        def add_rows(rb=rb, h=h):
            @plsc.parallel_loop(0, CHUNK)
            def _row(r):
                rr = rows[rb].at[r]
                pr = pos_v.at[h * CHUNK + r]

                @plsc.parallel_loop(0, DIM, step=LANES, unroll=8)
                def _slice(dd):
                    sl = pl.ds(dd, LANES)
                    plsc.addupdate(rr.at[sl], pr[sl])

        add_rows()
        o[c] = pltpu.async_copy(
            rows[rb],
            out_hbm.at[pl.ds(b * SEQ + pos_base + h * CHUNK, CHUNK)],
            osem[rb])
    for c in range(NCHUNK - 4, NCHUNK):
        o[c].wait()


@jax.jit
def kernel(input_ids, vocab_W, pos_W):
    ids = input_ids.astype(jnp.int32)
    mesh = plsc.VectorSubcoreMesh(core_axis_name="c", subcore_axis_name="s")
    run = pl.kernel(
        _body,
        out_type=jax.ShapeDtypeStruct((BATCH * SEQ, DIM), jnp.float32),
        mesh=mesh,
        scratch_types=(
            [pltpu.VMEM((BATCH, POSB), jnp.int32),
             pltpu.VMEM((POSB, DIM), jnp.float32)]
            + [pltpu.VMEM((CHUNK, DIM), jnp.float32) for _ in range(NRB)]
            + [pltpu.SemaphoreType.DMA for _ in range(1 + 2 * NRB)]
        ),
    )
    out = run(ids, vocab_W, pos_W)
    return out.reshape(BATCH, SEQ, DIM)
